# baseline (speedup 1.0000x reference)
import functools

import jax
import jax.numpy as jnp
from jax import lax
from jax.experimental import pallas as pl
from jax.experimental.pallas import tpu as pltpu
import jax.experimental.pallas.tpu_sc as plsc

N_CAT = 1_000_000

ROWS, COLS = 7840, 128
BLK_ROWS = 3920
GRID = ROWS // BLK_ROWS

NW = 32
CHUNK = 31256
STAGE_CHUNK = 31248


def _threefry_rounds(k0, k1, x0, x1):
    ks2 = k0 ^ k1 ^ jnp.uint32(0x1BD11BDA)
    x0 = x0 + k0
    x1 = x1 + k1
    ks = (k0, k1, ks2)
    rots = ((13, 15, 26, 6), (17, 29, 16, 24))
    for i in range(5):
        for r in rots[i % 2]:
            x0 = x0 + x1
            x1 = (x1 << jnp.uint32(r)) | (x1 >> jnp.uint32(32 - r))
            x1 = x1 ^ x0
        x0 = x0 + ks[(i + 1) % 3]
        x1 = x1 + ks[(i + 2) % 3] + jnp.uint32(i + 1)
    return x0, x1


def _tc_threefry_body(kd_ref, idx_ref):
    p = pl.program_id(0)
    k0, k1 = _threefry_rounds(kd_ref[0], kd_ref[1], jnp.uint32(0), jnp.uint32(1))

    row = lax.broadcasted_iota(jnp.uint32, (BLK_ROWS, COLS), 0)
    col = lax.broadcasted_iota(jnp.uint32, (BLK_ROWS, COLS), 1)
    base = (p * (BLK_ROWS * COLS)).astype(jnp.uint32)
    j = base + row * jnp.uint32(COLS) + col

    o0, o1 = _threefry_rounds(k0, k1, jnp.zeros_like(j), j)
    bits = o0 ^ o1

    a = (bits >> jnp.uint32(20)).astype(jnp.int32)
    b = (bits & jnp.uint32(0xFFFFF)).astype(jnp.int32)
    t = a * 48576 + b
    q = (t.astype(jnp.float32) * (1.0 / N_CAT)).astype(jnp.int32)
    r = t - q * N_CAT
    r = jnp.where(r < 0, r + N_CAT, r)
    r = jnp.where(r >= N_CAT, r - N_CAT, r)
    idx_ref[...] = r


def _tc_threefry(kd):
    return pl.pallas_call(
        _tc_threefry_body,
        grid=(GRID,),
        in_specs=[pl.BlockSpec(memory_space=pltpu.SMEM)],
        out_specs=pl.BlockSpec((BLK_ROWS, COLS), lambda p: (p, 0)),
        out_shape=jax.ShapeDtypeStruct((ROWS, COLS), jnp.int32),
    )(kd)


@functools.cache
def _sc_gather_fn():
    mesh = plsc.VectorSubcoreMesh(
        core_axis_name="c", subcore_axis_name="s", num_cores=2, num_subcores=16
    )

    @functools.partial(
        pl.kernel,
        out_type=jax.ShapeDtypeStruct((N_CAT,), jnp.float32),
        mesh=mesh,
        scratch_types=[
            pltpu.VMEM((CHUNK,), jnp.int32),
            pltpu.VMEM((CHUNK,), jnp.float32),
            pltpu.VMEM_SHARED((N_CAT,), jnp.float32),
            pltpu.SemaphoreType.DMA,
            pltpu.SemaphoreType.DMA,
        ],
    )
    def _sc_gather(table_hbm, idx_hbm, out_hbm, idx_v, rows_v, table_sp, sem, sem2):
        cid = lax.axis_index("c")
        sid = lax.axis_index("s")
        wid = sid * 2 + cid
        base = jnp.minimum(wid * CHUNK, N_CAT - CHUNK)
        base = pl.multiple_of(base, 8)
        pltpu.sync_copy(idx_hbm.at[pl.ds(base, CHUNK)], idx_v)
        for rnd in range(2):
            off = sid * (2 * STAGE_CHUNK) + rnd * STAGE_CHUNK
            off = pl.multiple_of(off, 8)
            pltpu.sync_copy(
                table_hbm.at[pl.ds(off, STAGE_CHUNK)],
                rows_v.at[pl.ds(0, STAGE_CHUNK)],
            )
            pltpu.sync_copy(
                rows_v.at[pl.ds(0, STAGE_CHUNK)],
                table_sp.at[pl.ds(off, STAGE_CHUNK)],
            )

        @pl.when(sid == 0)
        def _tail():
            tail_off = 16 * 2 * STAGE_CHUNK
            tail_n = N_CAT - 16 * 2 * STAGE_CHUNK
            pltpu.sync_copy(
                table_hbm.at[pl.ds(tail_off, tail_n)],
                rows_v.at[pl.ds(0, tail_n)],
            )
            pltpu.sync_copy(
                rows_v.at[pl.ds(0, tail_n)],
                table_sp.at[pl.ds(tail_off, tail_n)],
            )

        plsc.subcore_barrier()
        half = 15632
        rest = CHUNK - half
        d1 = pltpu.async_copy(
            table_sp.at[idx_v.at[pl.ds(0, half)]], rows_v.at[pl.ds(0, half)], sem
        )
        d2 = pltpu.async_copy(
            table_sp.at[idx_v.at[pl.ds(half, rest)]],
            rows_v.at[pl.ds(half, rest)],
            sem2,
        )
        d1.wait()
        o1 = pltpu.async_copy(
            rows_v.at[pl.ds(0, half)], out_hbm.at[pl.ds(base, half)], sem
        )
        d2.wait()
        pltpu.sync_copy(
            rows_v.at[pl.ds(half, rest)], out_hbm.at[pl.ds(base + half, rest)]
        )
        o1.wait()

    return _sc_gather


def kernel(key, categories):
    kd = jax.random.key_data(key).astype(jnp.uint32)
    idx = _tc_threefry(kd).reshape(-1)
    return _sc_gather_fn()(categories, idx)


# speedup vs baseline: 1.0013x; 1.0013x over previous
import functools

import jax
import jax.numpy as jnp
from jax import lax
from jax.experimental import pallas as pl
from jax.experimental.pallas import tpu as pltpu
import jax.experimental.pallas.tpu_sc as plsc

N_CAT = 1_000_000

ROWS, COLS = 7840, 128
BLK_ROWS = 1568
GRID = ROWS // BLK_ROWS

NW = 32
CHUNK = 31256
STAGE_CHUNK = 31248


def _threefry_rounds(k0, k1, x0, x1):
    ks2 = k0 ^ k1 ^ jnp.uint32(0x1BD11BDA)
    x0 = x0 + k0
    x1 = x1 + k1
    ks = (k0, k1, ks2)
    rots = ((13, 15, 26, 6), (17, 29, 16, 24))
    for i in range(5):
        for r in rots[i % 2]:
            x0 = x0 + x1
            x1 = (x1 << jnp.uint32(r)) | (x1 >> jnp.uint32(32 - r))
            x1 = x1 ^ x0
        x0 = x0 + ks[(i + 1) % 3]
        x1 = x1 + ks[(i + 2) % 3] + jnp.uint32(i + 1)
    return x0, x1


def _tc_threefry_body(kd_ref, idx_ref):
    p = pl.program_id(0)
    k0, k1 = _threefry_rounds(kd_ref[0], kd_ref[1], jnp.uint32(0), jnp.uint32(1))

    row = lax.broadcasted_iota(jnp.uint32, (BLK_ROWS, COLS), 0)
    col = lax.broadcasted_iota(jnp.uint32, (BLK_ROWS, COLS), 1)
    base = (p * (BLK_ROWS * COLS)).astype(jnp.uint32)
    j = base + row * jnp.uint32(COLS) + col

    o0, o1 = _threefry_rounds(k0, k1, jnp.zeros_like(j), j)
    bits = o0 ^ o1

    a = (bits >> jnp.uint32(20)).astype(jnp.int32)
    b = (bits & jnp.uint32(0xFFFFF)).astype(jnp.int32)
    t = a * 48576 + b
    q = (t.astype(jnp.float32) * (1.0 / N_CAT)).astype(jnp.int32)
    r = t - q * N_CAT
    r = jnp.where(r < 0, r + N_CAT, r)
    r = jnp.where(r >= N_CAT, r - N_CAT, r)
    idx_ref[...] = r


def _tc_threefry(kd):
    return pl.pallas_call(
        _tc_threefry_body,
        grid=(GRID,),
        in_specs=[pl.BlockSpec(memory_space=pltpu.SMEM)],
        out_specs=pl.BlockSpec((BLK_ROWS, COLS), lambda p: (p, 0)),
        out_shape=jax.ShapeDtypeStruct((ROWS, COLS), jnp.int32),
    )(kd)


@functools.cache
def _sc_gather_fn():
    mesh = plsc.VectorSubcoreMesh(
        core_axis_name="c", subcore_axis_name="s", num_cores=2, num_subcores=16
    )

    @functools.partial(
        pl.kernel,
        out_type=jax.ShapeDtypeStruct((N_CAT,), jnp.float32),
        mesh=mesh,
        scratch_types=[
            pltpu.VMEM((CHUNK,), jnp.int32),
            pltpu.VMEM((CHUNK,), jnp.float32),
            pltpu.VMEM_SHARED((N_CAT,), jnp.float32),
            pltpu.SemaphoreType.DMA,
        ],
    )
    def _sc_gather(table_hbm, idx_hbm, out_hbm, idx_v, rows_v, table_sp, sem):
        cid = lax.axis_index("c")
        sid = lax.axis_index("s")
        wid = sid * 2 + cid
        base = jnp.minimum(wid * CHUNK, N_CAT - CHUNK)
        base = pl.multiple_of(base, 8)
        pltpu.sync_copy(idx_hbm.at[pl.ds(base, CHUNK)], idx_v)
        for rnd in range(2):
            off = sid * (2 * STAGE_CHUNK) + rnd * STAGE_CHUNK
            off = pl.multiple_of(off, 8)
            pltpu.sync_copy(
                table_hbm.at[pl.ds(off, STAGE_CHUNK)],
                rows_v.at[pl.ds(0, STAGE_CHUNK)],
            )
            pltpu.sync_copy(
                rows_v.at[pl.ds(0, STAGE_CHUNK)],
                table_sp.at[pl.ds(off, STAGE_CHUNK)],
            )

        @pl.when(sid == 0)
        def _tail():
            tail_off = 16 * 2 * STAGE_CHUNK
            tail_n = N_CAT - 16 * 2 * STAGE_CHUNK
            pltpu.sync_copy(
                table_hbm.at[pl.ds(tail_off, tail_n)],
                rows_v.at[pl.ds(0, tail_n)],
            )
            pltpu.sync_copy(
                rows_v.at[pl.ds(0, tail_n)],
                table_sp.at[pl.ds(tail_off, tail_n)],
            )

        plsc.subcore_barrier()
        pltpu.async_copy(table_sp.at[idx_v], rows_v, sem).wait()
        pltpu.sync_copy(rows_v, out_hbm.at[pl.ds(base, CHUNK)])

    return _sc_gather


def kernel(key, categories):
    kd = jax.random.key_data(key).astype(jnp.uint32)
    idx = _tc_threefry(kd).reshape(-1)
    return _sc_gather_fn()(categories, idx)
